# baseline (speedup 1.0000x reference)
import functools

import jax
import jax.numpy as jnp
from jax import lax
from jax.experimental import pallas as pl
from jax.experimental.pallas import tpu as pltpu
from jax.experimental.pallas import tpu_sc as plsc

MAX_LEN = 20
TOP_K = 5


def _mm_nt(a, b):
    return lax.dot_general(
        a, b, dimension_numbers=(((1,), (1,)), ((), ())),
        preferred_element_type=jnp.float32)


def _gather_embed(emb, tok_flat):
    BT = tok_flat.shape[0]
    Hd = emb.shape[1]
    NW = 32
    b_per_w = BT // NW
    CH = 80
    n_ch = b_per_w // CH
    mesh = plsc.VectorSubcoreMesh(core_axis_name="c", subcore_axis_name="s")

    @functools.partial(
        pl.kernel, mesh=mesh,
        out_type=jax.ShapeDtypeStruct((BT, Hd), jnp.float32),
        scratch_types=[
            pltpu.VMEM((b_per_w,), jnp.int32),
            pltpu.VMEM((CH, Hd), jnp.float32),
            pltpu.VMEM((CH, Hd), jnp.float32),
            pltpu.SemaphoreType.DMA,
            pltpu.SemaphoreType.DMA,
        ],
    )
    def k(tok_hbm, emb_hbm, out_hbm, idx_v, buf0, buf1, sem0, sem1):
        wid = lax.axis_index("s") * 2 + lax.axis_index("c")
        base = wid * b_per_w
        pltpu.sync_copy(tok_hbm.at[pl.ds(base, b_per_w)], idx_v)
        bufs = (buf0, buf1)
        sems = (sem0, sem1)
        pending = [None, None]
        pending[0] = pltpu.async_copy(
            emb_hbm.at[idx_v.at[pl.ds(0, CH)]], bufs[0], sems[0])
        for i in range(n_ch):
            cur = i % 2
            nxt = (i + 1) % 2
            if i + 1 < n_ch:
                pending[nxt] = pltpu.async_copy(
                    emb_hbm.at[idx_v.at[pl.ds((i + 1) * CH, CH)]],
                    bufs[nxt], sems[nxt])
            pending[cur].wait()
            pltpu.sync_copy(bufs[cur], out_hbm.at[pl.ds(base + i * CH, CH)])

    return k(tok_flat, emb)


def _lstm_encode(x_tbh, w_ih, w_hh, bias_row):
    T, Bsz, Hd = x_tbh.shape

    def body(x_ref, wih_ref, whh_ref, b_ref, out_ref, h_ref, c_ref):
        t = pl.program_id(0)

        @pl.when(t == 0)
        def _():
            h_ref[...] = jnp.zeros_like(h_ref)
            c_ref[...] = jnp.zeros_like(c_ref)

        x = x_ref[0]
        h = h_ref[...]
        c = c_ref[...]
        gates = _mm_nt(x, wih_ref[...]) + _mm_nt(h, whh_ref[...]) \
            + b_ref[0][None, :]
        i_g = gates[:, :Hd]
        f_g = gates[:, Hd:2 * Hd]
        g_g = gates[:, 2 * Hd:3 * Hd]
        o_g = gates[:, 3 * Hd:]
        c_new = jax.nn.sigmoid(f_g) * c + jax.nn.sigmoid(i_g) * jnp.tanh(g_g)
        h_new = jax.nn.sigmoid(o_g) * jnp.tanh(c_new)
        h_ref[...] = h_new
        c_ref[...] = c_new

        @pl.when(t == T - 1)
        def _():
            out_ref[...] = h_new

    return pl.pallas_call(
        body,
        grid=(T,),
        in_specs=[
            pl.BlockSpec((1, Bsz, Hd), lambda t: (t, 0, 0)),
            pl.BlockSpec((4 * Hd, Hd), lambda t: (0, 0)),
            pl.BlockSpec((4 * Hd, Hd), lambda t: (0, 0)),
            pl.BlockSpec((1, 4 * Hd), lambda t: (0, 0)),
        ],
        out_specs=pl.BlockSpec((Bsz, Hd), lambda t: (0, 0)),
        out_shape=jax.ShapeDtypeStruct((Bsz, Hd), jnp.float32),
        scratch_shapes=[
            pltpu.VMEM((Bsz, Hd), jnp.float32),
            pltpu.VMEM((Bsz, Hd), jnp.float32),
        ],
    )(x_tbh, w_ih, w_hh, bias_row)


def _sim_matmul(q, kn):
    Bsz, Hd = q.shape
    S = kn.shape[0]
    KB = 2048
    nk = S // KB

    def body(q_ref, k_ref, o_ref):
        o_ref[...] = _mm_nt(q_ref[...], k_ref[...])

    return pl.pallas_call(
        body,
        grid=(nk,),
        in_specs=[
            pl.BlockSpec((Bsz, Hd), lambda j: (0, 0)),
            pl.BlockSpec((KB, Hd), lambda j: (j, 0)),
        ],
        out_specs=pl.BlockSpec((Bsz, KB), lambda j: (0, j)),
        out_shape=jax.ShapeDtypeStruct((Bsz, S), jnp.float32),
    )(q, kn)


def _gru_hiddens(recalled, fc_in_w, fc_in_b_row, gw_ih, gw_hh, gb_ih_row,
                 gb_hh_row):
    Bsz, Hd = recalled.shape

    def body(rec_ref, fw_ref, fb_ref, wih_ref, whh_ref, bih_ref, bhh_ref,
             out_ref):
        hidden = _mm_nt(rec_ref[...], fw_ref[...]) + fb_ref[0][None, :]
        inputs = jnp.zeros((Bsz, Hd), jnp.float32)
        for t in range(MAX_LEN):
            gi = _mm_nt(inputs, wih_ref[...]) + bih_ref[0][None, :]
            gh = _mm_nt(hidden, whh_ref[...]) + bhh_ref[0][None, :]
            r = jax.nn.sigmoid(gi[:, :Hd] + gh[:, :Hd])
            z = jax.nn.sigmoid(gi[:, Hd:2 * Hd] + gh[:, Hd:2 * Hd])
            n = jnp.tanh(gi[:, 2 * Hd:] + r * gh[:, 2 * Hd:])
            hidden = (1.0 - z) * n + z * hidden
            out_ref[t] = hidden
            inputs = hidden

    return pl.pallas_call(
        body,
        out_shape=jax.ShapeDtypeStruct((MAX_LEN, Bsz, Hd), jnp.float32),
    )(recalled, fc_in_w, fc_in_b_row, gw_ih, gw_hh, gb_ih_row, gb_hh_row)


def _logits_and_max(hid, w_out, b2d):
    Bsz, Hd = hid.shape
    V = w_out.shape[0]
    VB = 2048
    NV = (V + VB - 1) // VB
    NEG = float("-inf")

    def body(h_ref, w_ref, b_ref, lg_ref, m_ref, ms_ref):
        v = pl.program_id(0)

        @pl.when(v == 0)
        def _():
            ms_ref[...] = jnp.full_like(ms_ref, NEG)

        logits = _mm_nt(h_ref[...], w_ref[...]) + b_ref[0][None, :]
        lg_ref[...] = logits
        col_ids = v * VB + lax.broadcasted_iota(jnp.int32, (Bsz, VB), 1)
        masked = jnp.where(col_ids < V, logits, NEG)
        mx = jnp.max(masked, axis=1, keepdims=True)
        m_new = jnp.maximum(ms_ref[:, 0:1], mx)
        ms_ref[...] = jnp.broadcast_to(m_new, (Bsz, 128))

        @pl.when(v == NV - 1)
        def _():
            m_ref[...] = ms_ref[...]

    return pl.pallas_call(
        body,
        grid=(NV,),
        in_specs=[
            pl.BlockSpec((Bsz, Hd), lambda v: (0, 0)),
            pl.BlockSpec((VB, Hd), lambda v: (v, 0)),
            pl.BlockSpec((8, VB), lambda v: (0, v)),
        ],
        out_specs=[
            pl.BlockSpec((Bsz, VB), lambda v: (0, v)),
            pl.BlockSpec((Bsz, 128), lambda v: (0, 0)),
        ],
        out_shape=[
            jax.ShapeDtypeStruct((Bsz, V), jnp.float32),
            jax.ShapeDtypeStruct((Bsz, 128), jnp.float32),
        ],
        scratch_shapes=[pltpu.VMEM((Bsz, 128), jnp.float32)],
    )(hid, w_out, b2d)


def _select_top5(logits, m2d, l2d):
    Bsz, V = logits.shape
    VB = 2048
    NV = (V + VB - 1) // VB
    K = 8
    NEG = float("-inf")

    def body(lg_ref, m_ref, l_ref, tv_ref, ti_ref, sv_ref, si_ref):
        v = pl.program_id(0)

        @pl.when(v == 0)
        def _():
            sv_ref[...] = jnp.full_like(sv_ref, NEG)
            si_ref[...] = jnp.zeros_like(si_ref)

        s = lg_ref[...] - m_ref[:, 0:1]
        vq = s - l_ref[:, 0:1]
        col_ids = v * VB + lax.broadcasted_iota(jnp.int32, (Bsz, VB), 1)
        vq = jnp.where(col_ids < V, vq, NEG)

        NC = VB // 128
        vq3 = vq.reshape(Bsz, NC, 128)
        i3l = lax.broadcasted_iota(jnp.int32, (Bsz, NC, 128), 2)
        i2c = lax.broadcasted_iota(jnp.int32, (Bsz, NC), 1)
        i1l = lax.broadcasted_iota(jnp.int32, (Bsz, 128), 1)
        cmax = jnp.max(vq3, axis=2)
        cidx = jnp.min(jnp.where(vq3 == cmax[:, :, None], i3l, 2**30),
                       axis=2)
        blk_v = []
        blk_i = []
        for _ in range(TOP_K):
            bm = jnp.max(cmax, axis=1, keepdims=True)
            csel = jnp.min(jnp.where(cmax == bm, i2c, 2**30), axis=1,
                           keepdims=True)
            li = jnp.min(jnp.where(i2c == csel, cidx, 2**30), axis=1,
                         keepdims=True)
            blk_v.append(bm)
            blk_i.append(v * VB + csel * 128 + li)
            hit3 = (i2c[:, :, None] == csel[:, :, None]) & \
                (i3l == li[:, :, None])
            vq3 = jnp.where(hit3, NEG, vq3)
            xsel = jnp.max(jnp.where(i2c[:, :, None] == csel[:, :, None],
                                     vq3, NEG), axis=1)
            nmax = jnp.max(xsel, axis=1, keepdims=True)
            nidx = jnp.min(jnp.where(xsel == nmax, i1l, 2**30), axis=1,
                           keepdims=True)
            cmax = jnp.where(i2c == csel, nmax, cmax)
            cidx = jnp.where(i2c == csel, nidx, cidx)

        cand_v = jnp.concatenate([sv_ref[...]] + blk_v, axis=1)
        cand_i = jnp.concatenate([si_ref[...]] + blk_i, axis=1)
        pos_iota = lax.broadcasted_iota(jnp.int32, cand_v.shape, 1)
        new_v = []
        new_i = []
        for _ in range(TOP_K):
            m = jnp.max(cand_v, axis=1, keepdims=True)
            pos = jnp.min(jnp.where(cand_v == m, pos_iota, 2**30),
                          axis=1, keepdims=True)
            hit = pos_iota == pos
            sel = jnp.max(jnp.where(hit, cand_i, -1), axis=1, keepdims=True)
            new_v.append(m)
            new_i.append(sel)
            cand_v = jnp.where(hit, NEG, cand_v)
        pad_v = jnp.full((Bsz, K - TOP_K), NEG, jnp.float32)
        pad_i = jnp.zeros((Bsz, K - TOP_K), jnp.int32)
        sv_new = jnp.concatenate(new_v + [pad_v], axis=1)
        si_new = jnp.concatenate(new_i + [pad_i], axis=1)
        sv_ref[...] = sv_new
        si_ref[...] = si_new

        @pl.when(v == NV - 1)
        def _():
            tv_ref[...] = sv_new
            ti_ref[...] = si_new

    return pl.pallas_call(
        body,
        grid=(NV,),
        in_specs=[
            pl.BlockSpec((Bsz, VB), lambda v: (0, v)),
            pl.BlockSpec((Bsz, 128), lambda v: (0, 0)),
            pl.BlockSpec((Bsz, 128), lambda v: (0, 0)),
        ],
        out_specs=[
            pl.BlockSpec((Bsz, K), lambda v: (0, 0)),
            pl.BlockSpec((Bsz, K), lambda v: (0, 0)),
        ],
        out_shape=[
            jax.ShapeDtypeStruct((Bsz, K), jnp.float32),
            jax.ShapeDtypeStruct((Bsz, K), jnp.int32),
        ],
        scratch_shapes=[
            pltpu.VMEM((Bsz, K), jnp.float32),
            pltpu.VMEM((Bsz, K), jnp.int32),
        ],
    )(logits, m2d, l2d)


def kernel(tokens, emb, lstm_W_ih, lstm_W_hh, lstm_b_ih, lstm_b_hh,
           mem_keys, mem_weights, fc_in_W, fc_in_b, gru_W_ih, gru_W_hh,
           gru_b_ih, gru_b_hh, fc_out_W, fc_out_b):
    Bsz, T = tokens.shape
    V, Hd = emb.shape

    tok_flat = tokens.astype(jnp.int32).T.reshape(-1)
    x = _gather_embed(emb, tok_flat).reshape(T, Bsz, Hd)

    lstm_bias = (lstm_b_ih + lstm_b_hh).reshape(1, 4 * Hd)
    h_last = _lstm_encode(x, lstm_W_ih, lstm_W_hh, lstm_bias)
    encoded = h_last / (jnp.linalg.norm(h_last, axis=-1, keepdims=True)
                        + 1e-8)

    q = encoded / (jnp.linalg.norm(encoded, axis=-1, keepdims=True) + 1e-8)
    kn = mem_keys / (jnp.linalg.norm(mem_keys, axis=-1, keepdims=True)
                     + 1e-8)
    sim = _sim_matmul(q, kn)
    attn = jax.nn.softmax(sim, axis=-1)
    recalled = attn @ mem_weights

    hiddens = _gru_hiddens(
        recalled, fc_in_W, fc_in_b.reshape(1, Hd), gru_W_ih, gru_W_hh,
        gru_b_ih.reshape(1, 3 * Hd), gru_b_hh.reshape(1, 3 * Hd))

    b2d = jnp.broadcast_to(fc_out_b, (8, V))
    tvs, tis = [], []
    for t in range(MAX_LEN):
        logits, m2d = _logits_and_max(hiddens[t], fc_out_W, b2d)
        m1 = m2d[:, 0:1]
        zsum = jnp.sum(jnp.exp(logits - m1), axis=-1, keepdims=True)
        l2d = jnp.broadcast_to(jnp.log(zsum), (Bsz, 128))
        tv, ti = _select_top5(logits, m2d, l2d)
        tvs.append(tv[:, :TOP_K])
        tis.append(ti[:, :TOP_K])
    top_p = jnp.stack(tvs)
    top_i = jnp.stack(tis)

    skey = jax.random.key(1234)
    keys = jax.vmap(lambda t: jax.random.fold_in(skey, t))(
        jnp.arange(MAX_LEN))
    sampled = jax.vmap(
        lambda k, l: jax.random.categorical(k, l, axis=-1))(keys, top_p)
    next_tok = jnp.take_along_axis(top_i, sampled[:, :, None], axis=2)[..., 0]
    decoded = next_tok.T
    return encoded, decoded


# speedup vs baseline: 1.4648x; 1.4648x over previous
import functools

import jax
import jax.numpy as jnp
from jax import lax
from jax.experimental import pallas as pl
from jax.experimental.pallas import tpu as pltpu
from jax.experimental.pallas import tpu_sc as plsc

MAX_LEN = 20
TOP_K = 5


def _mm_nt(a, b):
    return lax.dot_general(
        a, b, dimension_numbers=(((1,), (1,)), ((), ())),
        preferred_element_type=jnp.float32)


def _gather_embed(emb, tok_flat):
    BT = tok_flat.shape[0]
    Hd = emb.shape[1]
    NW = 32
    b_per_w = BT // NW
    CH = 80
    n_ch = b_per_w // CH
    mesh = plsc.VectorSubcoreMesh(core_axis_name="c", subcore_axis_name="s")

    @functools.partial(
        pl.kernel, mesh=mesh,
        out_type=jax.ShapeDtypeStruct((BT, Hd), jnp.float32),
        scratch_types=[
            pltpu.VMEM((b_per_w,), jnp.int32),
            pltpu.VMEM((CH, Hd), jnp.float32),
            pltpu.VMEM((CH, Hd), jnp.float32),
            pltpu.SemaphoreType.DMA,
            pltpu.SemaphoreType.DMA,
        ],
    )
    def k(tok_hbm, emb_hbm, out_hbm, idx_v, buf0, buf1, sem0, sem1):
        wid = lax.axis_index("s") * 2 + lax.axis_index("c")
        base = wid * b_per_w
        pltpu.sync_copy(tok_hbm.at[pl.ds(base, b_per_w)], idx_v)
        bufs = (buf0, buf1)
        sems = (sem0, sem1)
        pending = [None, None]
        pending[0] = pltpu.async_copy(
            emb_hbm.at[idx_v.at[pl.ds(0, CH)]], bufs[0], sems[0])
        for i in range(n_ch):
            cur = i % 2
            nxt = (i + 1) % 2
            if i + 1 < n_ch:
                pending[nxt] = pltpu.async_copy(
                    emb_hbm.at[idx_v.at[pl.ds((i + 1) * CH, CH)]],
                    bufs[nxt], sems[nxt])
            pending[cur].wait()
            pltpu.sync_copy(bufs[cur], out_hbm.at[pl.ds(base + i * CH, CH)])

    return k(tok_flat, emb)


def _lstm_encode(x_tbh, w_ih, w_hh, bias_row):
    T, Bsz, Hd = x_tbh.shape

    def body(x_ref, wih_ref, whh_ref, b_ref, out_ref, h_ref, c_ref):
        t = pl.program_id(0)

        @pl.when(t == 0)
        def _():
            h_ref[...] = jnp.zeros_like(h_ref)
            c_ref[...] = jnp.zeros_like(c_ref)

        x = x_ref[0]
        h = h_ref[...]
        c = c_ref[...]
        gates = _mm_nt(x, wih_ref[...]) + _mm_nt(h, whh_ref[...]) \
            + b_ref[0][None, :]
        i_g = gates[:, :Hd]
        f_g = gates[:, Hd:2 * Hd]
        g_g = gates[:, 2 * Hd:3 * Hd]
        o_g = gates[:, 3 * Hd:]
        c_new = jax.nn.sigmoid(f_g) * c + jax.nn.sigmoid(i_g) * jnp.tanh(g_g)
        h_new = jax.nn.sigmoid(o_g) * jnp.tanh(c_new)
        h_ref[...] = h_new
        c_ref[...] = c_new

        @pl.when(t == T - 1)
        def _():
            out_ref[...] = h_new

    return pl.pallas_call(
        body,
        grid=(T,),
        in_specs=[
            pl.BlockSpec((1, Bsz, Hd), lambda t: (t, 0, 0)),
            pl.BlockSpec((4 * Hd, Hd), lambda t: (0, 0)),
            pl.BlockSpec((4 * Hd, Hd), lambda t: (0, 0)),
            pl.BlockSpec((1, 4 * Hd), lambda t: (0, 0)),
        ],
        out_specs=pl.BlockSpec((Bsz, Hd), lambda t: (0, 0)),
        out_shape=jax.ShapeDtypeStruct((Bsz, Hd), jnp.float32),
        scratch_shapes=[
            pltpu.VMEM((Bsz, Hd), jnp.float32),
            pltpu.VMEM((Bsz, Hd), jnp.float32),
        ],
    )(x_tbh, w_ih, w_hh, bias_row)


def _sim_matmul(q, kn):
    Bsz, Hd = q.shape
    S = kn.shape[0]
    KB = 2048
    nk = S // KB

    def body(q_ref, k_ref, o_ref):
        o_ref[...] = _mm_nt(q_ref[...], k_ref[...])

    return pl.pallas_call(
        body,
        grid=(nk,),
        in_specs=[
            pl.BlockSpec((Bsz, Hd), lambda j: (0, 0)),
            pl.BlockSpec((KB, Hd), lambda j: (j, 0)),
        ],
        out_specs=pl.BlockSpec((Bsz, KB), lambda j: (0, j)),
        out_shape=jax.ShapeDtypeStruct((Bsz, S), jnp.float32),
    )(q, kn)


def _gru_hiddens(recalled, fc_in_w, fc_in_b_row, gw_ih, gw_hh, gb_ih_row,
                 gb_hh_row):
    Bsz, Hd = recalled.shape

    def body(rec_ref, fw_ref, fb_ref, wih_ref, whh_ref, bih_ref, bhh_ref,
             out_ref):
        hidden = _mm_nt(rec_ref[...], fw_ref[...]) + fb_ref[0][None, :]
        inputs = jnp.zeros((Bsz, Hd), jnp.float32)
        for t in range(MAX_LEN):
            gi = _mm_nt(inputs, wih_ref[...]) + bih_ref[0][None, :]
            gh = _mm_nt(hidden, whh_ref[...]) + bhh_ref[0][None, :]
            r = jax.nn.sigmoid(gi[:, :Hd] + gh[:, :Hd])
            z = jax.nn.sigmoid(gi[:, Hd:2 * Hd] + gh[:, Hd:2 * Hd])
            n = jnp.tanh(gi[:, 2 * Hd:] + r * gh[:, 2 * Hd:])
            hidden = (1.0 - z) * n + z * hidden
            out_ref[t] = hidden
            inputs = hidden

    return pl.pallas_call(
        body,
        out_shape=jax.ShapeDtypeStruct((MAX_LEN, Bsz, Hd), jnp.float32),
    )(recalled, fc_in_w, fc_in_b_row, gw_ih, gw_hh, gb_ih_row, gb_hh_row)


def _logits_and_max(hid, w_out, b2d):
    Bsz, Hd = hid.shape
    V = w_out.shape[0]
    VB = 2048
    NV = (V + VB - 1) // VB
    NEG = float("-inf")

    def body(h_ref, w_ref, b_ref, lg_ref, m_ref, ms_ref):
        v = pl.program_id(0)

        @pl.when(v == 0)
        def _():
            ms_ref[...] = jnp.full_like(ms_ref, NEG)

        logits = _mm_nt(h_ref[...], w_ref[...]) + b_ref[0][None, :]
        lg_ref[...] = logits
        col_ids = v * VB + lax.broadcasted_iota(jnp.int32, (Bsz, VB), 1)
        masked = jnp.where(col_ids < V, logits, NEG)
        mx = jnp.max(masked, axis=1, keepdims=True)
        m_new = jnp.maximum(ms_ref[:, 0:1], mx)
        ms_ref[...] = jnp.broadcast_to(m_new, (Bsz, 128))

        @pl.when(v == NV - 1)
        def _():
            m_ref[...] = ms_ref[...]

    return pl.pallas_call(
        body,
        grid=(NV,),
        in_specs=[
            pl.BlockSpec((Bsz, Hd), lambda v: (0, 0)),
            pl.BlockSpec((VB, Hd), lambda v: (v, 0)),
            pl.BlockSpec((8, VB), lambda v: (0, v)),
        ],
        out_specs=[
            pl.BlockSpec((Bsz, VB), lambda v: (0, v)),
            pl.BlockSpec((Bsz, 128), lambda v: (0, 0)),
        ],
        out_shape=[
            jax.ShapeDtypeStruct((Bsz, V), jnp.float32),
            jax.ShapeDtypeStruct((Bsz, 128), jnp.float32),
        ],
        scratch_shapes=[pltpu.VMEM((Bsz, 128), jnp.float32)],
    )(hid, w_out, b2d)


def _select_top5(logits, m2d, l2d):
    Bsz, V = logits.shape
    VB = 2048
    NV = (V + VB - 1) // VB
    K = 8
    NEG = float("-inf")

    def body(lg_ref, m_ref, l_ref, tv_ref, ti_ref, sv_ref, si_ref):
        v = pl.program_id(0)

        @pl.when(v == 0)
        def _():
            sv_ref[...] = jnp.full_like(sv_ref, NEG)
            si_ref[...] = jnp.zeros_like(si_ref)

        s = lg_ref[...] - m_ref[:, 0:1]
        vq = s - l_ref[:, 0:1]
        col_ids = v * VB + lax.broadcasted_iota(jnp.int32, (Bsz, VB), 1)
        vq = jnp.where(col_ids < V, vq, NEG)
        cand_v = jnp.concatenate([sv_ref[...], vq], axis=1)
        cand_i = jnp.concatenate([si_ref[...], col_ids], axis=1)
        pos_iota = lax.broadcasted_iota(jnp.int32, cand_v.shape, 1)
        new_v = []
        new_i = []
        for _ in range(TOP_K):
            m = jnp.max(cand_v, axis=1, keepdims=True)
            pos = jnp.min(jnp.where(cand_v == m, pos_iota, 2**30),
                          axis=1, keepdims=True)
            hit = pos_iota == pos
            sel = jnp.max(jnp.where(hit, cand_i, -1), axis=1, keepdims=True)
            new_v.append(m)
            new_i.append(sel)
            cand_v = jnp.where(hit, NEG, cand_v)
        pad_v = jnp.full((Bsz, K - TOP_K), NEG, jnp.float32)
        pad_i = jnp.zeros((Bsz, K - TOP_K), jnp.int32)
        sv_new = jnp.concatenate(new_v + [pad_v], axis=1)
        si_new = jnp.concatenate(new_i + [pad_i], axis=1)
        sv_ref[...] = sv_new
        si_ref[...] = si_new

        @pl.when(v == NV - 1)
        def _():
            tv_ref[...] = sv_new
            ti_ref[...] = si_new

    return pl.pallas_call(
        body,
        grid=(NV,),
        in_specs=[
            pl.BlockSpec((Bsz, VB), lambda v: (0, v)),
            pl.BlockSpec((Bsz, 128), lambda v: (0, 0)),
            pl.BlockSpec((Bsz, 128), lambda v: (0, 0)),
        ],
        out_specs=[
            pl.BlockSpec((Bsz, K), lambda v: (0, 0)),
            pl.BlockSpec((Bsz, K), lambda v: (0, 0)),
        ],
        out_shape=[
            jax.ShapeDtypeStruct((Bsz, K), jnp.float32),
            jax.ShapeDtypeStruct((Bsz, K), jnp.int32),
        ],
        scratch_shapes=[
            pltpu.VMEM((Bsz, K), jnp.float32),
            pltpu.VMEM((Bsz, K), jnp.int32),
        ],
    )(logits, m2d, l2d)


def kernel(tokens, emb, lstm_W_ih, lstm_W_hh, lstm_b_ih, lstm_b_hh,
           mem_keys, mem_weights, fc_in_W, fc_in_b, gru_W_ih, gru_W_hh,
           gru_b_ih, gru_b_hh, fc_out_W, fc_out_b):
    Bsz, T = tokens.shape
    V, Hd = emb.shape

    tok_flat = tokens.astype(jnp.int32).T.reshape(-1)
    x = _gather_embed(emb, tok_flat).reshape(T, Bsz, Hd)

    lstm_bias = (lstm_b_ih + lstm_b_hh).reshape(1, 4 * Hd)
    h_last = _lstm_encode(x, lstm_W_ih, lstm_W_hh, lstm_bias)
    encoded = h_last / (jnp.linalg.norm(h_last, axis=-1, keepdims=True)
                        + 1e-8)

    q = encoded / (jnp.linalg.norm(encoded, axis=-1, keepdims=True) + 1e-8)
    kn = mem_keys / (jnp.linalg.norm(mem_keys, axis=-1, keepdims=True)
                     + 1e-8)
    sim = _sim_matmul(q, kn)
    attn = jax.nn.softmax(sim, axis=-1)
    recalled = attn @ mem_weights

    hiddens = _gru_hiddens(
        recalled, fc_in_W, fc_in_b.reshape(1, Hd), gru_W_ih, gru_W_hh,
        gru_b_ih.reshape(1, 3 * Hd), gru_b_hh.reshape(1, 3 * Hd))

    b2d = jnp.broadcast_to(fc_out_b, (8, V))
    tvs, tis = [], []
    for t in range(MAX_LEN):
        logits, m2d = _logits_and_max(hiddens[t], fc_out_W, b2d)
        m1 = m2d[:, 0:1]
        zsum = jnp.sum(jnp.exp(logits - m1), axis=-1, keepdims=True)
        l2d = jnp.broadcast_to(jnp.log(zsum), (Bsz, 128))
        tv, ti = _select_top5(logits, m2d, l2d)
        tvs.append(tv[:, :TOP_K])
        tis.append(ti[:, :TOP_K])
    top_p = jnp.stack(tvs)
    top_i = jnp.stack(tis)

    skey = jax.random.key(1234)
    keys = jax.vmap(lambda t: jax.random.fold_in(skey, t))(
        jnp.arange(MAX_LEN))
    sampled = jax.vmap(
        lambda k, l: jax.random.categorical(k, l, axis=-1))(keys, top_p)
    next_tok = jnp.take_along_axis(top_i, sampled[:, :, None], axis=2)[..., 0]
    decoded = next_tok.T
    return encoded, decoded
